# baseline (speedup 1.0000x reference)
import jax
import jax.numpy as jnp
from jax.experimental import pallas as pl
from jax.experimental.pallas import tpu as pltpu

_B, _T, _D = 8, 4096, 32
_K = 1024
_NQ = 2
_KW = 5
_CHUNK = 1024
_NCHUNK = _T // _CHUNK
_LOSS_SCALE = 1.25 / (_B * _T * _D)
_INV_SQRT2 = 0.7071067811865476


def _split3(cb):
    b1 = cb.astype(jnp.bfloat16)
    r1 = cb - b1.astype(jnp.float32)
    b2 = r1.astype(jnp.bfloat16)
    r2 = r1 - b2.astype(jnp.float32)
    b3 = r2.astype(jnp.bfloat16)
    return jnp.concatenate([b1, b2, b3], axis=1)


def _vq_conv_body(x_ref, cb_ref, w_ref,
                  out_ref, q_ref, loss_ref, idx_ref,
                  qpad_ref):
    b = pl.program_id(0)
    cb0 = cb_ref[0]
    cb1 = cb_ref[1]
    cb0n = jnp.sum(cb0 * cb0, axis=1)[None, :]
    cb1n = jnp.sum(cb1 * cb1, axis=1)[None, :]
    cb0t = cb0.astype(jnp.bfloat16).T
    cb1t = cb1.astype(jnp.bfloat16).T
    cb0s = _split3(cb0)
    cb1s = _split3(cb1)

    qpad_ref[0:2, :] = jnp.zeros((2, _D), jnp.float32)
    qpad_ref[_T + 2:_T + 4, :] = jnp.zeros((2, _D), jnp.float32)

    loss_acc = jnp.float32(0.0)
    iota = jax.lax.broadcasted_iota(jnp.int32, (_CHUNK, _K), 1)
    for c in range(_NCHUNK):
        x = x_ref[0, pl.ds(c * _CHUNK, _CHUNK), :]

        xn = jnp.sum(x * x, axis=1, keepdims=True)
        s = jnp.dot(x.astype(jnp.bfloat16), cb0t,
                    preferred_element_type=jnp.float32)
        d = (xn - 2.0 * s) + cb0n
        idx0 = jnp.argmin(d, axis=1)[:, None].astype(jnp.int32)
        oh = jnp.where(iota == idx0, 1.0, 0.0).astype(jnp.bfloat16)
        g = jnp.dot(oh, cb0s, preferred_element_type=jnp.float32)
        q0 = (g[:, 0:_D] + g[:, _D:2 * _D]) + g[:, 2 * _D:3 * _D]
        r = x - q0

        rn = jnp.sum(r * r, axis=1, keepdims=True)
        s = jnp.dot(r.astype(jnp.bfloat16), cb1t,
                    preferred_element_type=jnp.float32)
        d = (rn - 2.0 * s) + cb1n
        idx1 = jnp.argmin(d, axis=1)[:, None].astype(jnp.int32)
        oh = jnp.where(iota == idx1, 1.0, 0.0).astype(jnp.bfloat16)
        g = jnp.dot(oh, cb1s, preferred_element_type=jnp.float32)
        q1 = (g[:, 0:_D] + g[:, _D:2 * _D]) + g[:, 2 * _D:3 * _D]
        r2 = r - q1

        quant = q0 + q1
        loss_acc += jnp.sum(r * r) + jnp.sum(r2 * r2)
        q_ref[0, pl.ds(c * _CHUNK, _CHUNK), :] = quant
        qpad_ref[pl.ds(2 + c * _CHUNK, _CHUNK), :] = quant
        idx_ref[0, pl.ds(c * _CHUNK, _CHUNK), 0:1] = idx0
        idx_ref[0, pl.ds(c * _CHUNK, _CHUNK), 1:2] = idx1

    acc = jnp.dot(qpad_ref[pl.ds(0, _T), :].astype(jnp.bfloat16),
                  w_ref[0].astype(jnp.bfloat16),
                  preferred_element_type=jnp.float32)
    for k in range(1, _KW):
        acc = acc + jnp.dot(qpad_ref[pl.ds(k, _T), :].astype(jnp.bfloat16),
                            w_ref[k].astype(jnp.bfloat16),
                            preferred_element_type=jnp.float32)
    out_ref[0] = 0.5 * acc * (1.0 + jax.lax.erf(acc * _INV_SQRT2))

    prev = jnp.where(b == 0, jnp.zeros((1, 1), jnp.float32), loss_ref[0:1, 0:1])
    total = prev + loss_acc
    loss_ref[0:1, 0:1] = jnp.where(b == _B - 1, total * _LOSS_SCALE, total)


def kernel(inputs, codebooks, conv_w):
    out, quant, loss, idx = pl.pallas_call(
        _vq_conv_body,
        grid=(_B,),
        in_specs=[
            pl.BlockSpec((1, _T, _D), lambda b: (b, 0, 0)),
            pl.BlockSpec((_NQ, _K, _D), lambda b: (0, 0, 0)),
            pl.BlockSpec((_KW, _D, _D), lambda b: (0, 0, 0)),
        ],
        out_specs=(
            pl.BlockSpec((1, _T, _D), lambda b: (b, 0, 0)),
            pl.BlockSpec((1, _T, _D), lambda b: (b, 0, 0)),
            pl.BlockSpec((1, 1), lambda b: (0, 0)),
            pl.BlockSpec((1, _T, _NQ), lambda b: (b, 0, 0)),
        ),
        out_shape=(
            jax.ShapeDtypeStruct((_B, _T, _D), jnp.float32),
            jax.ShapeDtypeStruct((_B, _T, _D), jnp.float32),
            jax.ShapeDtypeStruct((1, 1), jnp.float32),
            jax.ShapeDtypeStruct((_B, _T, _NQ), jnp.int32),
        ),
        scratch_shapes=[pltpu.VMEM((_T + 4, _D), jnp.float32)],
    )(inputs, codebooks, conv_w)
    return (out, quant, loss[0, 0], jnp.transpose(idx, (2, 0, 1)))


# speedup vs baseline: 2.1827x; 2.1827x over previous
import jax
import jax.numpy as jnp
from jax.experimental import pallas as pl
from jax.experimental.pallas import tpu as pltpu

_B, _T, _D = 8, 4096, 32
_K = 1024
_NQ = 2
_KW = 5
_CHUNK = 1024
_NCHUNK = _T // _CHUNK
_LOSS_SCALE = 1.25 / (_B * _T * _D)
_INV_SQRT2 = 0.7071067811865476


def _split3t(cbt):
    b1 = cbt.astype(jnp.bfloat16)
    r1 = cbt - b1.astype(jnp.float32)
    b2 = r1.astype(jnp.bfloat16)
    r2 = r1 - b2.astype(jnp.float32)
    b3 = r2.astype(jnp.bfloat16)
    return jnp.concatenate([b1, b2, b3], axis=0)


def _vq_conv_body(xt_ref, cb_ref, w_ref,
                  out_ref, q_ref, loss_ref, idx_ref,
                  qpad_ref):
    b = pl.program_id(0)
    cb0 = cb_ref[0]
    cb1 = cb_ref[1]
    cb0n = jnp.sum(cb0 * cb0, axis=1, keepdims=True)
    cb1n = jnp.sum(cb1 * cb1, axis=1, keepdims=True)
    cb0b = cb0.astype(jnp.bfloat16)
    cb1b = cb1.astype(jnp.bfloat16)
    cb0st = _split3t(cb0.T)
    cb1st = _split3t(cb1.T)
    iota_col = jax.lax.broadcasted_iota(jnp.int32, (_K, 1), 0).astype(jnp.float32)

    qpad_ref[0:2, :] = jnp.zeros((2, _D), jnp.float32)
    qpad_ref[_T + 2:_T + 4, :] = jnp.zeros((2, _D), jnp.float32)

    loss_acc = jnp.float32(0.0)
    for c in range(_NCHUNK):
        xt = xt_ref[0, :, pl.ds(c * _CHUNK, _CHUNK)]

        xn = jnp.sum(xt * xt, axis=0, keepdims=True)
        s = jnp.dot(cb0b, xt.astype(jnp.bfloat16),
                    preferred_element_type=jnp.float32)
        d = (xn - 2.0 * s) + cb0n
        m = jnp.min(d, axis=0, keepdims=True)
        idx0 = jnp.min(jnp.where(d == m, iota_col, jnp.float32(_K)),
                       axis=0, keepdims=True)
        oh = (iota_col == idx0).astype(jnp.bfloat16)
        g = jnp.dot(cb0st, oh, preferred_element_type=jnp.float32)
        q0 = (g[0:_D] + g[_D:2 * _D]) + g[2 * _D:3 * _D]
        r = xt - q0

        rn = jnp.sum(r * r, axis=0, keepdims=True)
        s = jnp.dot(cb1b, r.astype(jnp.bfloat16),
                    preferred_element_type=jnp.float32)
        d = (rn - 2.0 * s) + cb1n
        m = jnp.min(d, axis=0, keepdims=True)
        idx1 = jnp.min(jnp.where(d == m, iota_col, jnp.float32(_K)),
                       axis=0, keepdims=True)
        oh = (iota_col == idx1).astype(jnp.bfloat16)
        g = jnp.dot(cb1st, oh, preferred_element_type=jnp.float32)
        q1 = (g[0:_D] + g[_D:2 * _D]) + g[2 * _D:3 * _D]
        r2 = r - q1

        quant = (q0 + q1).T
        loss_acc += jnp.sum(r * r) + jnp.sum(r2 * r2)
        q_ref[0, pl.ds(c * _CHUNK, _CHUNK), :] = quant
        qpad_ref[pl.ds(2 + c * _CHUNK, _CHUNK), :] = quant
        idx_ref[0, 0:1, pl.ds(c * _CHUNK, _CHUNK)] = idx0.astype(jnp.int32)
        idx_ref[0, 1:2, pl.ds(c * _CHUNK, _CHUNK)] = idx1.astype(jnp.int32)

    acc = jnp.dot(qpad_ref[pl.ds(0, _T), :].astype(jnp.bfloat16),
                  w_ref[0].astype(jnp.bfloat16),
                  preferred_element_type=jnp.float32)
    for k in range(1, _KW):
        acc = acc + jnp.dot(qpad_ref[pl.ds(k, _T), :].astype(jnp.bfloat16),
                            w_ref[k].astype(jnp.bfloat16),
                            preferred_element_type=jnp.float32)
    out_ref[0] = 0.5 * acc * (1.0 + jax.lax.erf(acc * _INV_SQRT2))

    prev = jnp.where(b == 0, jnp.zeros((1, 1), jnp.float32), loss_ref[0:1, 0:1])
    total = prev + loss_acc
    loss_ref[0:1, 0:1] = jnp.where(b == _B - 1, total * _LOSS_SCALE, total)


def kernel(inputs, codebooks, conv_w):
    xt = jnp.transpose(inputs, (0, 2, 1))
    out, quant, loss, idx = pl.pallas_call(
        _vq_conv_body,
        grid=(_B,),
        in_specs=[
            pl.BlockSpec((1, _D, _T), lambda b: (b, 0, 0)),
            pl.BlockSpec((_NQ, _K, _D), lambda b: (0, 0, 0)),
            pl.BlockSpec((_KW, _D, _D), lambda b: (0, 0, 0)),
        ],
        out_specs=(
            pl.BlockSpec((1, _T, _D), lambda b: (b, 0, 0)),
            pl.BlockSpec((1, _T, _D), lambda b: (b, 0, 0)),
            pl.BlockSpec((1, 1), lambda b: (0, 0)),
            pl.BlockSpec((1, _NQ, _T), lambda b: (b, 0, 0)),
        ),
        out_shape=(
            jax.ShapeDtypeStruct((_B, _T, _D), jnp.float32),
            jax.ShapeDtypeStruct((_B, _T, _D), jnp.float32),
            jax.ShapeDtypeStruct((1, 1), jnp.float32),
            jax.ShapeDtypeStruct((_B, _NQ, _T), jnp.int32),
        ),
        scratch_shapes=[pltpu.VMEM((_T + 4, _D), jnp.float32)],
    )(xt, codebooks, conv_w)
    return (out, quant, loss[0, 0], jnp.transpose(idx, (1, 0, 2)))
